# baseline (speedup 1.0000x reference)
import functools

import jax
import jax.numpy as jnp
from jax import lax
from jax.experimental import pallas as pl
from jax.experimental.pallas import tpu as pltpu
from jax.experimental.pallas import tpu_sc as plsc

MAX_DEGREE = 512
HIDDEN_DIM = 128
N_NODES = 100000
N_EDGES = 1600000

NC = 2
NS = 16
NW = NC * NS
L = 16

EROWS = N_EDGES // 128
BR = 64
BW = BR * 128
NBLK = EROWS // BR
REM_ROWS = EROWS - NBLK * BR
NBLK_MAX = -(-NBLK // NW) + 1
HIST_PAD = 100352
HSLICE = HIST_PAD // NS
C = 64
D = 6
NPT = HIST_PAD // NW
CPT = NPT // C
TAIL_BASE = (N_NODES // C) * C
TAIL_N = N_NODES - TAIL_BASE
TAIL_LOCAL = (TAIL_BASE - (NW - 1) * NPT) // C
TAIL_PARITY = TAIL_LOCAL % D
NV = CPT + D - 1
NSPLIT = 2
HS = C // NSPLIT

TAB_ROWS = MAX_DEGREE + 1
OUT_TAB = 520
SH_TAB_ROWS = OUT_TAB + TAB_ROWS

_mesh = functools.partial(
    plsc.VectorSubcoreMesh,
    core_axis_name="c", subcore_axis_name="s",
    num_cores=NC, num_subcores=NS,
)


def _hist_body(edges_hbm, hist_hbm, zbuf, eb0, eb1, ones_v, rbuf,
               sh_src, sh_tgt, dsem, ssem):
    cid = lax.axis_index("c")
    sid = lax.axis_index("s")
    wid = sid * NC + cid

    @pl.loop(0, HSLICE // L)
    def _(i):
        zbuf[pl.ds(i * L, L)] = jnp.zeros((L,), jnp.int32)

    for j in range(512 // L):
        ones_v[pl.ds(j * L, L)] = jnp.ones((L,), jnp.int32)

    pltpu.sync_copy(zbuf, sh_src.at[pl.ds(sid * HSLICE, HSLICE)])
    pltpu.sync_copy(zbuf, sh_tgt.at[pl.ds(sid * HSLICE, HSLICE)])
    plsc.subcore_barrier()

    nfull = NBLK // NW
    n = jnp.where(wid < NBLK - nfull * NW, nfull + 1, nfull)
    ebufs = (eb0, eb1)

    for kind, sh in ((0, sh_src), (1, sh_tgt)):
        @pl.when(n > 0)
        def _():
            off0 = pl.multiple_of(wid * BW, BW)
            pltpu.async_copy(edges_hbm.at[kind, pl.ds(off0, BW)],
                             ebufs[0], dsem)

        @pl.loop(0, NBLK_MAX, step=2)
        def _(i):
            for p in range(2):
                iv = i + p

                @pl.when(iv < n)
                def _():
                    ebuf = ebufs[p]
                    pltpu.make_async_copy(
                        edges_hbm.at[kind, pl.ds(0, BW)], ebuf, dsem).wait()

                    @pl.when(iv + 1 < n)
                    def _():
                        off = pl.multiple_of((wid + (iv + 1) * NW) * BW, BW)
                        pltpu.async_copy(
                            edges_hbm.at[kind, pl.ds(off, BW)],
                            ebufs[1 - p], dsem)

                    for j in range(BW // 512):
                        pltpu.async_copy(
                            ones_v, sh.at[ebuf.at[pl.ds(j * 512, 512)]],
                            ssem, add=True)
                    for j in range(BW // 512):
                        pltpu.make_async_copy(
                            ones_v, sh.at[ebuf.at[pl.ds(0, 512)]],
                            ssem).wait()

        @pl.when(wid < REM_ROWS)
        def _():
            roff = pl.multiple_of((NBLK * BR + wid) * 128, 128)
            pltpu.sync_copy(edges_hbm.at[kind, pl.ds(roff, 128)], rbuf)
            pltpu.sync_copy(ones_v.at[pl.ds(0, 128)], sh.at[rbuf], add=True)

    plsc.subcore_barrier()

    for kind, sh in ((0, sh_src), (1, sh_tgt)):
        src_sl = pl.ds(sid * HSLICE, HSLICE)
        doff = pl.multiple_of((2 * cid + kind) * HIST_PAD + sid * HSLICE,
                              HSLICE)
        pltpu.sync_copy(sh.at[src_sl], hist_hbm.at[pl.ds(doff, HSLICE)])


_hist_call = pl.kernel(
    _hist_body,
    out_type=jax.ShapeDtypeStruct((2 * NC * HIST_PAD,), jnp.int32),
    mesh=_mesh(),
    scratch_types=[
        pltpu.VMEM((HSLICE,), jnp.int32),
        pltpu.VMEM((BW,), jnp.int32),
        pltpu.VMEM((BW,), jnp.int32),
        pltpu.VMEM((512,), jnp.int32),
        pltpu.VMEM((128,), jnp.int32),
        pltpu.VMEM_SHARED((HIST_PAD,), jnp.int32),
        pltpu.VMEM_SHARED((HIST_PAD,), jnp.int32),
        pltpu.SemaphoreType.DMA,
        pltpu.SemaphoreType.DMA,
    ],
)


def _lookup_body(in_emb, out_emb, hist_hbm, off_hbm, out_hbm,
                 hstage, idx_all, rows, offv, sh_tab,
                 hsem, gsem, osem):
    cid = lax.axis_index("c")
    sid = lax.axis_index("s")
    wid = sid * NC + cid

    pltpu.sync_copy(off_hbm, offv)
    off_vec = offv[...]

    nbase = pl.multiple_of(wid * NPT, NPT)
    for j in range(4):
        pltpu.async_copy(hist_hbm.at[pl.ds(j * HIST_PAD + nbase, NPT)],
                         hstage.at[pl.ds(j * NPT, NPT)], hsem)

    for base, tab in ((0, in_emb), (OUT_TAB, out_emb)):
        @pl.when(sid < 12)
        def _():
            r0 = pl.multiple_of(sid * 40, 8)
            pltpu.sync_copy(tab.at[pl.ds(r0, 40)],
                            sh_tab.at[pl.ds(base + r0, 40)])

        @pl.when(sid == 12)
        def _():
            pltpu.sync_copy(tab.at[pl.ds(480, TAB_ROWS - 480)],
                            sh_tab.at[pl.ds(base + 480, TAB_ROWS - 480)])

    for j in range(4):
        pltpu.make_async_copy(hist_hbm.at[pl.ds(0, NPT)],
                              hstage.at[pl.ds(0, NPT)], hsem).wait()

    @pl.loop(0, CPT)
    def _(i):
        for j in range(C // L):
            pos = i * C + j * L
            v = hstage[pl.ds(1 * NPT + pos, L)] + \
                hstage[pl.ds(3 * NPT + pos, L)] + off_vec
            idx_all[pl.ds(i * 2 * C + j * L, L)] = (
                jnp.minimum(jnp.maximum(v, 0), MAX_DEGREE))
            w = hstage[pl.ds(0 * NPT + pos, L)] + \
                hstage[pl.ds(2 * NPT + pos, L)] + off_vec
            idx_all[pl.ds(i * 2 * C + C + j * L, L)] = (
                jnp.minimum(jnp.maximum(w, 0), MAX_DEGREE) + OUT_TAB)

    plsc.subcore_barrier()

    @pl.loop(0, NV, step=D)
    def _(i):
        for p in range(D):
            iv = i + p

            @pl.when(iv < CPT)
            def _():
                @pl.when(iv >= D)
                def _():
                    for _h in range(NSPLIT):
                        pltpu.make_async_copy(
                            rows[p].at[pl.ds(0, HS)],
                            out_hbm.at[pl.ds(0, HS)], osem[p]).wait()

                ioff = pl.multiple_of(iv * 2 * C, 2 * C)
                pltpu.async_copy(
                    sh_tab.at[idx_all.at[pl.ds(ioff, 2 * C)]],
                    rows[p], gsem[p])

            ivm = iv - (D - 1)
            q = (p + 1) % D

            @pl.when((ivm >= 0) & (ivm < CPT))
            def _():
                base = wid * NPT + ivm * C
                pltpu.make_async_copy(
                    sh_tab.at[idx_all.at[pl.ds(0, 2 * C)]],
                    rows[q], gsem[q]).wait()

                for h in range(NSPLIT):
                    @pl.loop(h * HS, (h + 1) * HS)
                    def _(r):
                        for j in range(HIDDEN_DIM // L):
                            sl = pl.ds(j * L, L)
                            rows[q][r, sl] = (rows[q][r, sl]
                                              + rows[q][C + r, sl])

                    @pl.when(base + (h + 1) * HS <= N_NODES)
                    def _():
                        ob = pl.multiple_of(base + h * HS, HS)
                        pltpu.async_copy(rows[q].at[pl.ds(h * HS, HS)],
                                         out_hbm.at[pl.ds(ob, HS)], osem[q])

    for p in range(D):
        nst_last = TAIL_N // HS if p == TAIL_PARITY else 0

        @pl.when(wid == NW - 1)
        def _():
            for _h in range(nst_last):
                pltpu.make_async_copy(
                    rows[p].at[pl.ds(0, HS)],
                    out_hbm.at[pl.ds(0, HS)], osem[p]).wait()

        @pl.when(wid != NW - 1)
        def _():
            for _h in range(NSPLIT):
                pltpu.make_async_copy(
                    rows[p].at[pl.ds(0, HS)],
                    out_hbm.at[pl.ds(0, HS)], osem[p]).wait()


_lookup_call = pl.kernel(
    _lookup_body,
    out_type=jax.ShapeDtypeStruct((N_NODES, HIDDEN_DIM), jnp.float32),
    mesh=_mesh(),
    scratch_types=[
        pltpu.VMEM((4 * NPT,), jnp.int32),
        pltpu.VMEM((2 * NPT,), jnp.int32),
        [pltpu.VMEM((2 * C, HIDDEN_DIM), jnp.float32) for _ in range(D)],
        pltpu.VMEM((L,), jnp.int32),
        pltpu.VMEM_SHARED((SH_TAB_ROWS + 7, HIDDEN_DIM), jnp.float32),
        pltpu.SemaphoreType.DMA,
        [pltpu.SemaphoreType.DMA for _ in range(D)],
        [pltpu.SemaphoreType.DMA for _ in range(D)],
    ],
)


def kernel(edge_index, n_nodes, in_embed, out_embed):
    off = (jnp.asarray(n_nodes) - N_NODES).astype(jnp.int32)
    off_v = jnp.full((L,), off, jnp.int32)
    hist = _hist_call(edge_index)
    return _lookup_call(in_embed, out_embed, hist, off_v)


# speedup vs baseline: 1.0157x; 1.0157x over previous
import functools

import jax
import jax.numpy as jnp
from jax import lax
from jax.experimental import pallas as pl
from jax.experimental.pallas import tpu as pltpu
from jax.experimental.pallas import tpu_sc as plsc

MAX_DEGREE = 512
HIDDEN_DIM = 128
N_NODES = 100000
N_EDGES = 1600000

NC = 2
NS = 16
NW = NC * NS
L = 16

EROWS = N_EDGES // 128
BR = 32
BW = BR * 128
NBLK = EROWS // BR
REM_ROWS = EROWS - NBLK * BR
NBLK_MAX = -(-NBLK // NW) + 1
HIST_PAD = 100352
HSLICE = HIST_PAD // NS
C = 64
D = 6
NPT = HIST_PAD // NW
CPT = NPT // C
TAIL_BASE = (N_NODES // C) * C
TAIL_N = N_NODES - TAIL_BASE
TAIL_LOCAL = (TAIL_BASE - (NW - 1) * NPT) // C
TAIL_PARITY = TAIL_LOCAL % D
NV = CPT + D - 1
NSPLIT = 2
HS = C // NSPLIT

TAB_ROWS = MAX_DEGREE + 1
OUT_TAB = 520
SH_TAB_ROWS = OUT_TAB + TAB_ROWS

_mesh = functools.partial(
    plsc.VectorSubcoreMesh,
    core_axis_name="c", subcore_axis_name="s",
    num_cores=NC, num_subcores=NS,
)


def _hist_body(edges_hbm, hist_hbm, zbuf, eb0, eb1, ones_v, rbuf,
               sh_src, sh_tgt, dsem, ssem):
    cid = lax.axis_index("c")
    sid = lax.axis_index("s")
    wid = sid * NC + cid

    @pl.loop(0, HSLICE // L)
    def _(i):
        zbuf[pl.ds(i * L, L)] = jnp.zeros((L,), jnp.int32)

    for j in range(512 // L):
        ones_v[pl.ds(j * L, L)] = jnp.ones((L,), jnp.int32)

    pltpu.sync_copy(zbuf, sh_src.at[pl.ds(sid * HSLICE, HSLICE)])
    pltpu.sync_copy(zbuf, sh_tgt.at[pl.ds(sid * HSLICE, HSLICE)])
    plsc.subcore_barrier()

    nfull = NBLK // NW
    n = jnp.where(wid < NBLK - nfull * NW, nfull + 1, nfull)
    ebufs = (eb0, eb1)

    for kind, sh in ((0, sh_src), (1, sh_tgt)):
        @pl.when(n > 0)
        def _():
            off0 = pl.multiple_of(wid * BW, BW)
            pltpu.async_copy(edges_hbm.at[kind, pl.ds(off0, BW)],
                             ebufs[0], dsem)

        @pl.loop(0, NBLK_MAX, step=2)
        def _(i):
            for p in range(2):
                iv = i + p

                @pl.when(iv < n)
                def _():
                    ebuf = ebufs[p]
                    pltpu.make_async_copy(
                        edges_hbm.at[kind, pl.ds(0, BW)], ebuf, dsem).wait()

                    @pl.when(iv + 1 < n)
                    def _():
                        off = pl.multiple_of((wid + (iv + 1) * NW) * BW, BW)
                        pltpu.async_copy(
                            edges_hbm.at[kind, pl.ds(off, BW)],
                            ebufs[1 - p], dsem)

                    for j in range(BW // 512):
                        pltpu.async_copy(
                            ones_v, sh.at[ebuf.at[pl.ds(j * 512, 512)]],
                            ssem, add=True)
                    for j in range(BW // 512):
                        pltpu.make_async_copy(
                            ones_v, sh.at[ebuf.at[pl.ds(0, 512)]],
                            ssem).wait()

        @pl.when(wid < REM_ROWS)
        def _():
            roff = pl.multiple_of((NBLK * BR + wid) * 128, 128)
            pltpu.sync_copy(edges_hbm.at[kind, pl.ds(roff, 128)], rbuf)
            pltpu.sync_copy(ones_v.at[pl.ds(0, 128)], sh.at[rbuf], add=True)

    plsc.subcore_barrier()

    for kind, sh in ((0, sh_src), (1, sh_tgt)):
        src_sl = pl.ds(sid * HSLICE, HSLICE)
        doff = pl.multiple_of((2 * cid + kind) * HIST_PAD + sid * HSLICE,
                              HSLICE)
        pltpu.sync_copy(sh.at[src_sl], hist_hbm.at[pl.ds(doff, HSLICE)])


_hist_call = pl.kernel(
    _hist_body,
    out_type=jax.ShapeDtypeStruct((2 * NC * HIST_PAD,), jnp.int32),
    mesh=_mesh(),
    scratch_types=[
        pltpu.VMEM((HSLICE,), jnp.int32),
        pltpu.VMEM((BW,), jnp.int32),
        pltpu.VMEM((BW,), jnp.int32),
        pltpu.VMEM((512,), jnp.int32),
        pltpu.VMEM((128,), jnp.int32),
        pltpu.VMEM_SHARED((HIST_PAD,), jnp.int32),
        pltpu.VMEM_SHARED((HIST_PAD,), jnp.int32),
        pltpu.SemaphoreType.DMA,
        pltpu.SemaphoreType.DMA,
    ],
)


def _lookup_body(in_emb, out_emb, hist_hbm, off_hbm, out_hbm,
                 hstage, idx_all, rows, offv, sh_tab,
                 hsem, gsem, osem):
    cid = lax.axis_index("c")
    sid = lax.axis_index("s")
    wid = sid * NC + cid

    pltpu.sync_copy(off_hbm, offv)
    off_vec = offv[...]

    nbase = pl.multiple_of(wid * NPT, NPT)
    for j in range(4):
        pltpu.async_copy(hist_hbm.at[pl.ds(j * HIST_PAD + nbase, NPT)],
                         hstage.at[pl.ds(j * NPT, NPT)], hsem)

    for base, tab in ((0, in_emb), (OUT_TAB, out_emb)):
        @pl.when(sid < 12)
        def _():
            r0 = pl.multiple_of(sid * 40, 8)
            pltpu.sync_copy(tab.at[pl.ds(r0, 40)],
                            sh_tab.at[pl.ds(base + r0, 40)])

        @pl.when(sid == 12)
        def _():
            pltpu.sync_copy(tab.at[pl.ds(480, TAB_ROWS - 480)],
                            sh_tab.at[pl.ds(base + 480, TAB_ROWS - 480)])

    for j in range(4):
        pltpu.make_async_copy(hist_hbm.at[pl.ds(0, NPT)],
                              hstage.at[pl.ds(0, NPT)], hsem).wait()

    @pl.loop(0, CPT)
    def _(i):
        for j in range(C // L):
            pos = i * C + j * L
            v = hstage[pl.ds(1 * NPT + pos, L)] + \
                hstage[pl.ds(3 * NPT + pos, L)] + off_vec
            idx_all[pl.ds(i * 2 * C + j * L, L)] = (
                jnp.minimum(jnp.maximum(v, 0), MAX_DEGREE))
            w = hstage[pl.ds(0 * NPT + pos, L)] + \
                hstage[pl.ds(2 * NPT + pos, L)] + off_vec
            idx_all[pl.ds(i * 2 * C + C + j * L, L)] = (
                jnp.minimum(jnp.maximum(w, 0), MAX_DEGREE) + OUT_TAB)

    plsc.subcore_barrier()

    @pl.loop(0, NV, step=D)
    def _(i):
        for p in range(D):
            iv = i + p

            @pl.when(iv < CPT)
            def _():
                @pl.when(iv >= D)
                def _():
                    for _h in range(NSPLIT):
                        pltpu.make_async_copy(
                            rows[p].at[pl.ds(0, HS)],
                            out_hbm.at[pl.ds(0, HS)], osem[p]).wait()

                ioff = pl.multiple_of(iv * 2 * C, 2 * C)
                pltpu.async_copy(
                    sh_tab.at[idx_all.at[pl.ds(ioff, 2 * C)]],
                    rows[p], gsem[p])

            ivm = iv - (D - 1)
            q = (p + 1) % D

            @pl.when((ivm >= 0) & (ivm < CPT))
            def _():
                base = wid * NPT + ivm * C
                pltpu.make_async_copy(
                    sh_tab.at[idx_all.at[pl.ds(0, 2 * C)]],
                    rows[q], gsem[q]).wait()

                for h in range(NSPLIT):
                    @pl.loop(h * HS, (h + 1) * HS)
                    def _(r):
                        for j in range(HIDDEN_DIM // L):
                            sl = pl.ds(j * L, L)
                            rows[q][r, sl] = (rows[q][r, sl]
                                              + rows[q][C + r, sl])

                    @pl.when(base + (h + 1) * HS <= N_NODES)
                    def _():
                        ob = pl.multiple_of(base + h * HS, HS)
                        pltpu.async_copy(rows[q].at[pl.ds(h * HS, HS)],
                                         out_hbm.at[pl.ds(ob, HS)], osem[q])

    for p in range(D):
        nst_last = TAIL_N // HS if p == TAIL_PARITY else 0

        @pl.when(wid == NW - 1)
        def _():
            for _h in range(nst_last):
                pltpu.make_async_copy(
                    rows[p].at[pl.ds(0, HS)],
                    out_hbm.at[pl.ds(0, HS)], osem[p]).wait()

        @pl.when(wid != NW - 1)
        def _():
            for _h in range(NSPLIT):
                pltpu.make_async_copy(
                    rows[p].at[pl.ds(0, HS)],
                    out_hbm.at[pl.ds(0, HS)], osem[p]).wait()


_lookup_call = pl.kernel(
    _lookup_body,
    out_type=jax.ShapeDtypeStruct((N_NODES, HIDDEN_DIM), jnp.float32),
    mesh=_mesh(),
    scratch_types=[
        pltpu.VMEM((4 * NPT,), jnp.int32),
        pltpu.VMEM((2 * NPT,), jnp.int32),
        [pltpu.VMEM((2 * C, HIDDEN_DIM), jnp.float32) for _ in range(D)],
        pltpu.VMEM((L,), jnp.int32),
        pltpu.VMEM_SHARED((SH_TAB_ROWS + 7, HIDDEN_DIM), jnp.float32),
        pltpu.SemaphoreType.DMA,
        [pltpu.SemaphoreType.DMA for _ in range(D)],
        [pltpu.SemaphoreType.DMA for _ in range(D)],
    ],
)


def kernel(edge_index, n_nodes, in_embed, out_embed):
    off = (jnp.asarray(n_nodes) - N_NODES).astype(jnp.int32)
    off_v = jnp.full((L,), off, jnp.int32)
    hist = _hist_call(edge_index)
    return _lookup_call(in_embed, out_embed, hist, off_v)
